# baseline (speedup 1.0000x reference)
import jax
import jax.numpy as jnp
from jax import lax
from jax.experimental import pallas as pl
from jax.experimental.pallas import tpu as pltpu
from jax.experimental.pallas import tpu_sc as plsc

_BATCH = 16384
_VOCAB = 1000000
_DIM = 3
_NC, _NS = 2, 16
_NW = _NC * _NS
_CHUNK = 128
_CPW = _BATCH // (_NW * _CHUNK)
_BPW = _BATCH // _NW
_NTAB = 3
_SUB = _CHUNK // 16
_QSCALE = 4096.0


def _body(idx_hbm, w0, w1, w2, out_hbm, idx_v, fidx_v, vals_v, out_v, sem):
    c = lax.axis_index("c")
    s = lax.axis_index("s")
    w = c * _NS + s

    pltpu.sync_copy(idx_hbm.at[pl.ds(w * _BPW, _BPW)], idx_v)

    for j in range(_CPW):
        for cc in range(_SUB):
            v = idx_v[pl.ds(j * _CHUNK + cc * 16, 16)] >> 1
            for d in range(_DIM):
                fidx_v[d * _CPW + j, pl.ds(cc * 16, 16)] = v + d * (_VOCAB // 2)

    copies = []
    for t, tbl in enumerate((w0, w1, w2)):
        for dj in range(_DIM * _CPW):
            copies.append(
                pltpu.async_copy(
                    tbl.at[fidx_v.at[dj]], vals_v.at[t * _DIM * _CPW + dj], sem
                )
            )
    for cp in copies:
        cp.wait()

    accs = [jnp.zeros((16,), jnp.float32) for _ in range(_NTAB * _DIM)]
    for t in range(_NTAB):
        for d in range(_DIM):
            for j in range(_CPW):
                row = t * _DIM * _CPW + d * _CPW + j
                for cc in range(_SUB):
                    v = idx_v[pl.ds(j * _CHUNK + cc * 16, 16)]
                    peven = (v & 1) == 0
                    wd = vals_v[row, pl.ds(cc * 16, 16)]
                    val = jnp.where(peven, (wd << 16) >> 16, wd >> 16)
                    accs[t * _DIM + d] = (
                        accs[t * _DIM + d] + val.astype(jnp.float32)
                    )

    iota = lax.iota(jnp.int32, 16)
    _dnums = lax.GatherDimensionNumbers(
        offset_dims=(), collapsed_slice_dims=(0,), start_index_map=(0,)
    )

    def _shuffle(v, idx16):
        return lax.gather(
            v,
            idx16[:, None],
            _dnums,
            slice_sizes=(1,),
            mode=lax.GatherScatterMode.PROMISE_IN_BOUNDS,
        )

    def _lane_sum(v):
        for sh in (1, 2, 4, 8):
            v = v + _shuffle(v, jnp.bitwise_xor(iota, sh))
        return v

    part = jnp.zeros((16,), jnp.float32)
    for k2 in range(_NTAB * _DIM):
        part = jnp.where(iota == k2, _lane_sum(accs[k2]), part)
    scale = (jnp.where(
        iota < 3, 5.0, jnp.where(iota < 6, 10.0, jnp.where(iota < 9, 6.0, 0.0))
    ) * (1.0 / _QSCALE)).astype(jnp.float32)
    out_v[...] = part * scale

    pltpu.sync_copy(out_v, out_hbm.at[w])


_sc_call = pl.kernel(
    _body,
    out_type=jax.ShapeDtypeStruct((_NW, 16), jnp.float32),
    mesh=plsc.VectorSubcoreMesh(core_axis_name="c", subcore_axis_name="s"),
    scratch_types=[
        pltpu.VMEM((_BPW,), jnp.int32),
        pltpu.VMEM((_DIM * _CPW, _CHUNK), jnp.int32),
        pltpu.VMEM((_NTAB * _DIM * _CPW, _CHUNK), jnp.int32),
        pltpu.VMEM((16,), jnp.float32),
        pltpu.SemaphoreType.DMA,
    ],
)


def _packT(w):
    q = jnp.clip(jnp.round(w * _QSCALE), -32768, 32767).astype(jnp.int16)
    return jax.lax.bitcast_convert_type(
        q.T.reshape(_DIM * _VOCAB // 2, 2), jnp.int32
    )


@jax.jit
def kernel(e_input, W0, W1, W2):
    out = _sc_call(
        e_input.astype(jnp.int32), _packT(W0), _packT(W1), _packT(W2)
    )
    return out.sum(axis=0)[: _NTAB * _DIM].reshape(_NTAB, _DIM)


# speedup vs baseline: 24.8617x; 24.8617x over previous
import jax
import jax.numpy as jnp
from jax import lax
from jax.experimental import pallas as pl
from jax.experimental.pallas import tpu as pltpu
from jax.experimental.pallas import tpu_sc as plsc

_BATCH = 16384
_VOCAB = 1000000
_DIM = 3
_NC, _NS = 2, 16
_NW = _NC * _NS
_CHUNK = 128
_CPW = _BATCH // (_NW * _CHUNK)
_BPW = _BATCH // _NW
_NTAB = 3
_SUB = _CHUNK // 16


def _body(idx_hbm, w0, w1, w2, out_hbm, idx_v, fidx_v, vals_v, out_v, sem):
    c = lax.axis_index("c")
    s = lax.axis_index("s")
    w = c * _NS + s

    pltpu.sync_copy(idx_hbm.at[pl.ds(w * _BPW, _BPW)], idx_v)

    for j in range(_CPW):
        for cc in range(_SUB):
            v = idx_v[pl.ds(j * _CHUNK + cc * 16, 16)]
            for d in range(_DIM):
                fidx_v[d * _CPW + j, pl.ds(cc * 16, 16)] = v + d * _VOCAB

    copies = []
    for t, tbl in enumerate((w0, w1, w2)):
        for dj in range(_DIM * _CPW):
            copies.append(
                pltpu.async_copy(
                    tbl.at[fidx_v.at[dj]], vals_v.at[t * _DIM * _CPW + dj], sem
                )
            )
    for cp in copies:
        cp.wait()

    accs = [jnp.zeros((16,), jnp.float32) for _ in range(_NTAB * _DIM)]
    for t in range(_NTAB):
        for d in range(_DIM):
            for j in range(_CPW):
                row = t * _DIM * _CPW + d * _CPW + j
                for cc in range(_SUB):
                    accs[t * _DIM + d] = (
                        accs[t * _DIM + d] + vals_v[row, pl.ds(cc * 16, 16)]
                    )

    iota = lax.iota(jnp.int32, 16)
    _dnums = lax.GatherDimensionNumbers(
        offset_dims=(), collapsed_slice_dims=(0,), start_index_map=(0,)
    )

    def _shuffle(v, idx16):
        return lax.gather(
            v,
            idx16[:, None],
            _dnums,
            slice_sizes=(1,),
            mode=lax.GatherScatterMode.PROMISE_IN_BOUNDS,
        )

    def _lane_sum(v):
        for sh in (1, 2, 4, 8):
            v = v + _shuffle(v, jnp.bitwise_xor(iota, sh))
        return v

    part = jnp.zeros((16,), jnp.float32)
    for k2 in range(_NTAB * _DIM):
        part = jnp.where(iota == k2, _lane_sum(accs[k2]), part)
    scale = jnp.where(
        iota < 3, 5.0, jnp.where(iota < 6, 10.0, jnp.where(iota < 9, 6.0, 0.0))
    ).astype(jnp.float32)
    out_v[...] = part * scale

    pltpu.sync_copy(out_v, out_hbm.at[w])


_sc_call = pl.kernel(
    _body,
    out_type=jax.ShapeDtypeStruct((_NW, 16), jnp.float32),
    mesh=plsc.VectorSubcoreMesh(core_axis_name="c", subcore_axis_name="s"),
    scratch_types=[
        pltpu.VMEM((_BPW,), jnp.int32),
        pltpu.VMEM((_DIM * _CPW, _CHUNK), jnp.int32),
        pltpu.VMEM((_NTAB * _DIM * _CPW, _CHUNK), jnp.float32),
        pltpu.VMEM((16,), jnp.float32),
        pltpu.SemaphoreType.DMA,
    ],
)


@jax.jit
def kernel(e_input, W0, W1, W2):
    out = _sc_call(
        e_input.astype(jnp.int32),
        W0.T.reshape(_DIM * _VOCAB),
        W1.T.reshape(_DIM * _VOCAB),
        W2.T.reshape(_DIM * _VOCAB),
    )
    return out.sum(axis=0)[: _NTAB * _DIM].reshape(_NTAB, _DIM)
